# baseline (speedup 1.0000x reference)
import functools

import numpy as np
import jax
import jax.numpy as jnp
from jax import lax
from jax.experimental import pallas as pl
from jax.experimental.pallas import tpu as pltpu, tpu_sc as plsc

NUM_IN = 1024
NUM_OUT = 2048
WIDTH = 1024
BATCH = 8
OB = 128
BS = 2
PAIR = 2

SC_CORES = 2
SC_SUBCORES = 16
SC_LANES = 16
NW = SC_CORES * SC_SUBCORES

NG_PAIR = PAIR * NUM_OUT
G_PER_W = NG_PAIR // NW
KCH = 16
NCH_W = G_PER_W // KCH

NELEM_A = BS * 2 * NUM_OUT * NUM_IN
PER_W_A = NELEM_A // NW
TILE_A = 16384


def _threefry_keys():
    def tf(k1, k2, x0, x1):
        ks = [np.uint32(k1), np.uint32(k2)]
        ks.append(np.uint32(ks[0] ^ ks[1] ^ np.uint32(0x1BD11BDA)))
        rot = [(13, 15, 26, 6), (17, 29, 16, 24)]
        x = [np.uint32(x0), np.uint32(x1)]
        x[0] += ks[0]
        x[1] += ks[1]
        for i, (inj_a, inj_b) in enumerate([(1, 2), (2, 0), (0, 1), (1, 2), (2, 0)]):
            for r in rot[i % 2]:
                x[0] += x[1]
                x[1] = np.uint32((x[1] << np.uint32(r)) | (x[1] >> np.uint32(32 - r)))
                x[1] ^= x[0]
            x[0] += ks[inj_a]
            x[1] += ks[inj_b] + np.uint32(i + 1)
        return x[0], x[1]

    with np.errstate(over="ignore"):
        a0, b0 = tf(0, 1234, 0, 0)
        a1, b1 = tf(0, 1234, 0, 1)
    return (a0, a1), (b0, b1)


(_KS1_A, _KS2_A), (_KS1_B, _KS2_B) = _threefry_keys()
_TINY = np.float32(np.finfo(np.float32).tiny)
_EXP_ONE = np.uint32(0x3F800000)


def _bits(k1, k2, cnt_lo):
    u32 = jnp.uint32
    ks0 = u32(k1)
    ks1 = u32(k2)
    ks2 = u32(np.uint32(k1) ^ np.uint32(k2) ^ np.uint32(0x1BD11BDA))
    ks = (ks0, ks1, ks2)
    rot = ((13, 15, 26, 6), (17, 29, 16, 24))
    x0 = jnp.zeros_like(cnt_lo) + ks0
    x1 = cnt_lo + ks1
    for i, (inj_a, inj_b) in enumerate(((1, 2), (2, 0), (0, 1), (1, 2), (2, 0))):
        for r in rot[i % 2]:
            x0 = x0 + x1
            x1 = (x1 << u32(r)) | (x1 >> u32(32 - r))
            x1 = x0 ^ x1
        x0 = x0 + ks[inj_a]
        x1 = x1 + ks[inj_b] + u32(i + 1)
    return x0 ^ x1


def _uniform_from_bits(bits):
    fb = (bits >> jnp.uint32(9)) | jnp.uint32(_EXP_ONE)
    return jax.lax.bitcast_convert_type(fb, jnp.float32) - jnp.float32(1.0)


def _finish_uniform(f):
    return jnp.maximum(_TINY, f * (jnp.float32(1.0) - _TINY) + _TINY)


def _sc_u_body(u_hbm, out_v, iota_v):
    wid = lax.axis_index("s") * SC_CORES + lax.axis_index("c")
    base = wid * PER_W_A
    iota_v[...] = jax.lax.iota(jnp.int32, SC_LANES)

    def tile(t, _):
        tbase = base + t * TILE_A

        def slice_body(s, _):
            off = s * SC_LANES
            cnt = (jnp.broadcast_to(tbase + off, (SC_LANES,)) + iota_v[...]).astype(jnp.uint32)
            u = _finish_uniform(_uniform_from_bits(_bits(_KS1_A, _KS1_B, cnt)))
            out_v[pl.ds(off, SC_LANES)] = u
            return 0

        lax.fori_loop(0, TILE_A // SC_LANES, slice_body, 0, unroll=2)
        pltpu.sync_copy(out_v, u_hbm.at[pl.ds(tbase, TILE_A)])
        return 0

    lax.fori_loop(0, PER_W_A // TILE_A, tile, 0)


def _sc_uniforms():
    mesh = plsc.VectorSubcoreMesh(core_axis_name="c", subcore_axis_name="s")
    kfn = pl.kernel(
        _sc_u_body,
        out_type=jax.ShapeDtypeStruct((NELEM_A,), jnp.float32),
        mesh=mesh,
        scratch_types=[
            pltpu.VMEM((TILE_A,), jnp.float32),
            pltpu.VMEM((SC_LANES,), jnp.int32),
        ],
    )
    return kfn()


def _mask_body(p_ref, mask_ref):
    r = jax.lax.broadcasted_iota(jnp.int32, (16, 128), 0)
    c = jax.lax.broadcasted_iota(jnp.int32, (16, 128), 1)
    cnt = (r * 128 + c).astype(jnp.uint32)
    mf = _uniform_from_bits(_bits(_KS2_A, _KS2_B, cnt))
    mask_ref[...] = jnp.where(mf < p_ref[...], jnp.int32(-1), jnp.int32(0))


def _mask(p2d):
    return pl.pallas_call(
        _mask_body,
        out_shape=jax.ShapeDtypeStruct((16, 128), jnp.int32),
    )(p2d)


_CNT0 = (
    np.arange(2 * OB, dtype=np.int32)[:, None] * NUM_IN
    + np.arange(NUM_IN, dtype=np.int32)[None, :]
    + (np.arange(2 * OB, dtype=np.int32)[:, None] >= OB) * ((NUM_OUT - OB) * NUM_IN)
)


def _argmax_store(g, cols, idx0_ref, idx1_ref):
    m = jnp.max(g, axis=1, keepdims=True)
    idxc = jnp.min(jnp.where(g == m, cols, NUM_IN), axis=1, keepdims=True)
    idxc = idxc.astype(jnp.int32)
    idx0_ref[...] = idxc[:OB].reshape(1, OB, 1)
    idx1_ref[...] = idxc[OB:].reshape(1, OB, 1)


def _sample_compute_body(b0, cnt0_ref, logits_ref, idx0_ref, idx1_ref):
    b = pl.program_id(0) + b0
    ob = pl.program_id(1)

    logits = logits_ref[...].reshape(2 * OB, NUM_IN)
    cnt0 = cnt0_ref[...]
    cols = cnt0 & jnp.int32(NUM_IN - 1)
    base = (b * (2 * NUM_OUT) + ob * OB) * NUM_IN
    cnt = (cnt0 + base).astype(jnp.uint32)

    u = _finish_uniform(_uniform_from_bits(_bits(_KS1_A, _KS1_B, cnt)))
    g = -jnp.log(-jnp.log(u)) + logits
    _argmax_store(g, cols, idx0_ref, idx1_ref)


def _sample_compute(cnt0, adjacency_matrix_logits, b0):
    grid = (PAIR, NUM_OUT // OB)
    out_shapes = (
        jax.ShapeDtypeStruct((PAIR, NUM_OUT, 1), jnp.int32),
        jax.ShapeDtypeStruct((PAIR, NUM_OUT, 1), jnp.int32),
    )
    return pl.pallas_call(
        functools.partial(_sample_compute_body, b0),
        grid=grid,
        in_specs=[
            pl.BlockSpec((2 * OB, NUM_IN), lambda b, ob: (0, 0)),
            pl.BlockSpec((2, OB, NUM_IN), lambda b, ob: (0, ob, 0)),
        ],
        out_specs=(
            pl.BlockSpec((1, OB, 1), lambda b, ob: (b, ob, 0)),
            pl.BlockSpec((1, OB, 1), lambda b, ob: (b, ob, 0)),
        ),
        out_shape=out_shapes,
    )(cnt0, adjacency_matrix_logits)


def _sample_load_body(u_ref, logits_ref, idx0_ref, idx1_ref):
    logits = logits_ref[...].reshape(2 * OB, NUM_IN)
    cols = jax.lax.broadcasted_iota(jnp.int32, (2 * OB, NUM_IN), 1)
    u = u_ref[...].reshape(2 * OB, NUM_IN)
    g = -jnp.log(-jnp.log(u)) + logits
    _argmax_store(g, cols, idx0_ref, idx1_ref)


def _sample_load(u4d, adjacency_matrix_logits):
    grid = (BS, NUM_OUT // OB)
    out_shapes = (
        jax.ShapeDtypeStruct((BS, NUM_OUT, 1), jnp.int32),
        jax.ShapeDtypeStruct((BS, NUM_OUT, 1), jnp.int32),
    )
    return pl.pallas_call(
        _sample_load_body,
        grid=grid,
        in_specs=[
            pl.BlockSpec((1, 2, OB, NUM_IN), lambda b, ob: (b, 0, ob, 0)),
            pl.BlockSpec((2, OB, NUM_IN), lambda b, ob: (0, ob, 0)),
        ],
        out_specs=(
            pl.BlockSpec((1, OB, 1), lambda b, ob: (b, ob, 0)),
            pl.BlockSpec((1, OB, 1), lambda b, ob: (b, ob, 0)),
        ),
        out_shape=out_shapes,
    )(u4d, adjacency_matrix_logits)


def _sc_body(table_hbm, idx0_hbm, idx1_hbm, mask_hbm, out_hbm,
             i0a, i1a, i0b, i1b, mask_v, r0a, r1a, r0b, r1b, out_v,
             s0a, s1a, s0b, s1b):
    wid = lax.axis_index("s") * SC_CORES + lax.axis_index("c")
    base = wid * G_PER_W
    pltpu.sync_copy(mask_hbm, mask_v)

    bufs = ((i0a, i1a, r0a, r1a, s0a, s1a), (i0b, i1b, r0b, r1b, s0b, s1b))

    def start(c, buf):
        i0_v, i1_v, r0_v, r1_v, sem0, sem1 = buf
        g0 = base + c * KCH
        pltpu.sync_copy(idx0_hbm.at[pl.ds(g0, KCH)], i0_v)
        pltpu.sync_copy(idx1_hbm.at[pl.ds(g0, KCH)], i1_v)
        pltpu.make_async_copy(table_hbm.at[i0_v], r0_v, sem0).start()
        pltpu.make_async_copy(table_hbm.at[i1_v], r1_v, sem1).start()

    def finish(c, buf):
        i0_v, i1_v, r0_v, r1_v, sem0, sem1 = buf
        g0 = base + c * KCH
        pltpu.make_async_copy(table_hbm.at[i0_v], r0_v, sem0).wait()
        pltpu.make_async_copy(table_hbm.at[i1_v], r1_v, sem1).wait()
        mvec = mask_v[pl.ds(lax.rem(g0, NUM_OUT), KCH)]
        for g in range(KCH):
            inv = jnp.broadcast_to(mvec[g], (SC_LANES,))

            def inner(w, _):
                sl = pl.ds(w * SC_LANES, SC_LANES)
                out_v[g, sl] = (r0_v[g, sl] & r1_v[g, sl]) ^ inv
                return 0

            lax.fori_loop(0, WIDTH // SC_LANES, inner, 0, unroll=8)
        pltpu.sync_copy(out_v, out_hbm.at[pl.ds(g0, KCH)])

    start(0, bufs[0])

    def pair(c2, _):
        for par in range(2):
            c = c2 * 2 + par

            @pl.when(c + 1 < NCH_W)
            def _():
                start(c + 1, bufs[1 - par])

            finish(c, bufs[par])
        return 0

    lax.fori_loop(0, NCH_W // 2, pair, 0)


def _sc_gather_pair(table, idx0_flat, idx1_flat, mask_i32):
    mesh = plsc.VectorSubcoreMesh(core_axis_name="c", subcore_axis_name="s")
    kfn = pl.kernel(
        _sc_body,
        out_type=jax.ShapeDtypeStruct((NG_PAIR, WIDTH), jnp.int32),
        mesh=mesh,
        scratch_types=[
            pltpu.VMEM((KCH,), jnp.int32),
            pltpu.VMEM((KCH,), jnp.int32),
            pltpu.VMEM((KCH,), jnp.int32),
            pltpu.VMEM((KCH,), jnp.int32),
            pltpu.VMEM((NUM_OUT,), jnp.int32),
            pltpu.VMEM((KCH, WIDTH), jnp.int32),
            pltpu.VMEM((KCH, WIDTH), jnp.int32),
            pltpu.VMEM((KCH, WIDTH), jnp.int32),
            pltpu.VMEM((KCH, WIDTH), jnp.int32),
            pltpu.VMEM((KCH, WIDTH), jnp.int32),
            pltpu.SemaphoreType.DMA,
            pltpu.SemaphoreType.DMA,
            pltpu.SemaphoreType.DMA,
            pltpu.SemaphoreType.DMA,
        ],
    )
    return kfn(table, idx0_flat, idx1_flat, mask_i32)


def kernel(input_bitarrays, batch_size, adjacency_matrix_logits, invert_logits):
    cnt0 = jnp.asarray(_CNT0)
    p2d = jax.nn.sigmoid(invert_logits).reshape(16, 128)
    mask2d = _mask(p2d)
    mask_flat = mask2d.reshape(NUM_OUT)

    u_flat = _sc_uniforms()
    u4d = u_flat.reshape(BS, 2, NUM_OUT, NUM_IN)

    idx_pairs = [None] * 4
    funcs = [None] * 4

    def gather(pair_i, idx0, idx1):
        f = _sc_gather_pair(
            input_bitarrays,
            idx0.reshape(NG_PAIR),
            idx1.reshape(NG_PAIR),
            mask_flat,
        )
        funcs[pair_i] = f.reshape(PAIR, NUM_OUT, WIDTH)
        idx_pairs[pair_i] = (idx0, idx1)

    i0, i1 = _sample_compute(cnt0, adjacency_matrix_logits, 2)
    gather(1, i0, i1)
    i0, i1 = _sample_compute(cnt0, adjacency_matrix_logits, 4)
    gather(2, i0, i1)
    i0, i1 = _sample_load(u4d, adjacency_matrix_logits)
    gather(0, i0, i1)
    i0, i1 = _sample_compute(cnt0, adjacency_matrix_logits, 6)
    gather(3, i0, i1)

    func = jnp.concatenate(funcs, axis=0)
    idx0 = jnp.concatenate([p[0] for p in idx_pairs], axis=0)
    idx1 = jnp.concatenate([p[1] for p in idx_pairs], axis=0)
    connection_indices = jnp.concatenate([idx0, idx1], axis=2)
    invert_mask = mask_flat != 0
    return (func, connection_indices, invert_mask)
